# baseline (speedup 1.0000x reference)
import functools

import jax
import jax.numpy as jnp
from jax import lax
from jax.experimental import pallas as pl
from jax.experimental.pallas import tpu as pltpu
from jax.experimental.pallas import tpu_sc as plsc

_EN = 320000
_NN = 10000
_NPAD = 10240
_NC, _NS = 2, 16
_TE = 4000
_TN = 2000
_CHUNK = 128
_NW = _NC * _NS
_NCH = _EN // _CHUNK
_CPW = _NCH // _NW
_REM = _NCH % _NW
_RPT = _NPAD // _NS


def _ln(v, g, b):
    mu = jnp.mean(v, axis=-1, keepdims=True)
    var = jnp.mean((v - mu) ** 2, axis=-1, keepdims=True)
    return (v - mu) * lax.rsqrt(var + 1e-5) * g + b


def _leaky(v):
    return jnp.where(v >= 0, v, v * 0.01)


def _mlp(v, ws, cast=False):
    n = len(ws)
    for j, (w, b) in enumerate(ws):
        if cast:
            v = v.astype(jnp.bfloat16)
        v = jnp.dot(v, w, preferred_element_type=jnp.float32) + b
        if j < n - 1:
            v = _leaky(v)
    return v


def _edge_body(ea_ref, *refs):
    out_ref = refs[-1]
    w = [r[...] for r in refs[:-1]]
    enc = [(w[0], w[1]), (w[2], w[3]), (w[4], w[5])]
    mcat = [(w[6], w[7]), (w[8], w[9]), (w[10], w[11])]
    z = _mlp(ea_ref[...], enc)
    mu = jnp.mean(z, axis=-1, keepdims=True)
    var = jnp.mean((z - mu) ** 2, axis=-1, keepdims=True)
    t = (z - mu) * lax.rsqrt(var + 1e-5)
    out_ref[...] = _mlp(t, mcat)


def _edge_kernel(edge_attr, weights, half, ne):
    full = [
        pl.BlockSpec(a.shape, lambda i, nd=a.ndim: (0,) * nd) for a in weights
    ]
    nsteps = ne // _TE
    return pl.pallas_call(
        _edge_body,
        grid=(nsteps,),
        in_specs=[
            pl.BlockSpec((_TE, 16), lambda i, h=half, n=nsteps: (i + h * n, 0))
        ]
        + full,
        out_specs=pl.BlockSpec((_TE, 128), lambda i: (i, 0)),
        out_shape=jax.ShapeDtypeStruct((ne, 128), jnp.float32),
        compiler_params=pltpu.CompilerParams(
            dimension_semantics=("arbitrary",)
        ),
    )(edge_attr, *weights)


def _make_scatter_body(ne, off):
    nch = ne // _CHUNK
    cpw = nch // _NW
    rem = nch % _NW
    nfull = cpw - (cpw % 2)

    def scatter_body(
        msg_hbm, dst_hbm, zeros_hbm, out_hbm, idx_v, rows_v, sem0, sem1, acc_sh
    ):
        c = lax.axis_index("c")
        s = lax.axis_index("s")
        wid = c * _NS + s
        r0 = s * _RPT
        sems = (sem0, sem1)

        def start(j, b):
            base = (wid + j * _NW) * _CHUNK
            pltpu.async_copy(
                dst_hbm.at[pl.ds(off + base, _CHUNK)], idx_v.at[b], sems[b]
            )
            pltpu.async_copy(
                msg_hbm.at[pl.ds(base, _CHUNK)], rows_v.at[b], sems[b]
            )

        def wait(b):
            pltpu.make_async_copy(
                dst_hbm.at[pl.ds(0, _CHUNK)], idx_v.at[b], sems[b]
            ).wait()
            pltpu.make_async_copy(
                msg_hbm.at[pl.ds(0, _CHUNK)], rows_v.at[b], sems[b]
            ).wait()

        def chunk_sync(j):
            base = (wid + j * _NW) * _CHUNK
            pltpu.sync_copy(dst_hbm.at[pl.ds(off + base, _CHUNK)], idx_v.at[0])
            pltpu.sync_copy(msg_hbm.at[pl.ds(base, _CHUNK)], rows_v.at[0])
            pltpu.sync_copy(rows_v.at[0], acc_sh.at[idx_v.at[0]], add=True)

        for b in range(2):
            start(b, b)

        pltpu.sync_copy(
            zeros_hbm.at[pl.ds(r0, _RPT)], acc_sh.at[pl.ds(r0, _RPT)]
        )
        plsc.subcore_barrier()

        def body(it, carry):
            j0 = it * 2
            for b in range(2):
                j = j0 + b
                wait(b)
                pltpu.sync_copy(rows_v.at[b], acc_sh.at[idx_v.at[b]], add=True)

                @pl.when(j + 2 < nfull)
                def _():
                    start(j + 2, b)

            return carry

        lax.fori_loop(0, nfull // 2, body, 0)

        for j in range(nfull, cpw):
            chunk_sync(j)

        @pl.when(wid < rem)
        def _():
            chunk_sync(cpw)

        plsc.subcore_barrier()
        pltpu.sync_copy(
            acc_sh.at[pl.ds(r0, _RPT)], out_hbm.at[c, pl.ds(r0, _RPT)]
        )

    return scatter_body


@functools.cache
def _build_scatter_kernel(ne, off):
    return functools.partial(
        pl.kernel,
        out_type=jax.ShapeDtypeStruct((_NC, _NPAD, 128), jnp.float32),
        mesh=plsc.VectorSubcoreMesh(
            core_axis_name="c", subcore_axis_name="s", num_cores=_NC
        ),
        scratch_types=[
            pltpu.VMEM((2, _CHUNK), jnp.int32),
            pltpu.VMEM((2, _CHUNK, 128), jnp.float32),
            pltpu.SemaphoreType.DMA,
            pltpu.SemaphoreType.DMA,
            pltpu.VMEM_SHARED((_NPAD, 128), jnp.float32),
        ],
    )(_make_scatter_body(ne, off))


def _scatter_kernel(msg, dst, zeros, off=0):
    return _build_scatter_kernel(msg.shape[0], off)(msg, dst, zeros)


def _node_body(nparts, x_ref, *refs):
    p_refs = refs[:nparts]
    out_ref = refs[-1]
    w = [r[...] for r in refs[nparts:-1]]
    enc = [(w[0], w[1]), (w[2], w[3]), (w[4], w[5])]
    eg, eb = w[6], w[7]
    dec = [(w[8], w[9]), (w[10], w[11]), (w[12], w[13])]
    y = _ln(_mlp(x_ref[...], enc), eg, eb)
    p = sum(pr[0] + pr[1] for pr in p_refs)
    aggrs = [
        lax.slice_in_dim(p, 0, 64, axis=1),
        lax.slice_in_dim(p, 64, 128, axis=1),
    ]
    k = 14
    for i in range(2):
        ng, nb = w[k], w[k + 1]
        upd = [(w[k + 2], w[k + 3]), (w[k + 4], w[k + 5]), (w[k + 6], w[k + 7])]
        og, ob = w[k + 8], w[k + 9]
        k += 10
        h = _ln(jnp.concatenate([y, aggrs[i]], axis=-1), ng, nb)
        y = y + _ln(_mlp(h, upd), og, ob)
    out_ref[...] = _mlp(y, dec)


def _node_kernel(x, partials_list, weights):
    full = [
        pl.BlockSpec(a.shape, lambda i, nd=a.ndim: (0,) * nd) for a in weights
    ]
    pspecs = [
        pl.BlockSpec((_NC, _TN, 128), lambda i: (0, i, 0))
        for _ in partials_list
    ]
    return pl.pallas_call(
        functools.partial(_node_body, len(partials_list)),
        grid=(_NN // _TN,),
        in_specs=[pl.BlockSpec((_TN, 128), lambda i: (i, 0))]
        + pspecs
        + full,
        out_specs=pl.BlockSpec((_TN, 128), lambda i: (i, 0)),
        out_shape=jax.ShapeDtypeStruct((_NN, 128), jnp.float32),
        compiler_params=pltpu.CompilerParams(
            dimension_semantics=("arbitrary",)
        ),
    )(x, *partials_list, *weights)


def _flat_mlp(ps, wdtype=jnp.float32):
    out = []
    for wmat, bvec in ps:
        out.append(jnp.transpose(wmat).astype(wdtype))
        out.append(jnp.reshape(bvec, (1, -1)))
    return out


def _flat_ln(p):
    g, b = p
    return [jnp.reshape(g, (1, -1)), jnp.reshape(b, (1, -1))]


def kernel(x, edge_attr, params, edge_index):
    dst = edge_index[1].astype(jnp.int32)

    m0, m1 = params['layers'][0]['msg'], params['layers'][1]['msg']
    g, bn = params['enc_edge']['ln']
    w1t = jnp.concatenate(
        [jnp.transpose(m0[0][0]), jnp.transpose(m1[0][0])], axis=1
    )
    b1c = jnp.concatenate([m0[0][1], m1[0][1]])
    w1f = g[:, None] * w1t
    b1f = jnp.reshape(bn @ w1t + b1c, (1, -1))

    def _blockdiag(a, b):
        za = jnp.zeros_like(a)
        return jnp.block([[a, za], [za, b]])

    mcat = [(w1f, b1f)]
    for j in (1, 2):
        wd = _blockdiag(jnp.transpose(m0[j][0]), jnp.transpose(m1[j][0]))
        bd = jnp.reshape(jnp.concatenate([m0[j][1], m1[j][1]]), (1, -1))
        mcat.append((wd, bd))

    edge_w = _flat_mlp(params['enc_edge']['mlp']) + [
        a for wb in mcat for a in wb
    ]
    node_w = (
        _flat_mlp(params['enc']['mlp'])
        + _flat_ln(params['enc']['ln'])
        + _flat_mlp(params['dec'])
    )
    for lp in params['layers']:
        node_w += (
            _flat_ln(lp['norm']) + _flat_mlp(lp['upd']) + _flat_ln(lp['outer_ln'])
        )

    zeros = jnp.zeros((_NPAD, 128), jnp.float32)
    nsplit = 5
    eh = _EN // nsplit
    parts = []
    for q in range(nsplit):
        msg_q = _edge_kernel(edge_attr, edge_w, q, eh)
        parts.append(_scatter_kernel(msg_q, dst, zeros, q * eh))
    return _node_kernel(x, parts, node_w)


# speedup vs baseline: 1.0307x; 1.0307x over previous
import functools

import jax
import jax.numpy as jnp
from jax import lax
from jax.experimental import pallas as pl
from jax.experimental.pallas import tpu as pltpu
from jax.experimental.pallas import tpu_sc as plsc

_EN = 320000
_NN = 10000
_NPAD = 10240
_NC, _NS = 2, 16
_TE = 8000
_TN = 2000
_CHUNK = 128
_NW = _NC * _NS
_NCH = _EN // _CHUNK
_CPW = _NCH // _NW
_REM = _NCH % _NW
_RPT = _NPAD // _NS


def _ln(v, g, b):
    mu = jnp.mean(v, axis=-1, keepdims=True)
    var = jnp.mean((v - mu) ** 2, axis=-1, keepdims=True)
    return (v - mu) * lax.rsqrt(var + 1e-5) * g + b


def _leaky(v):
    return jnp.maximum(v, v * 0.01)


def _mlp(v, ws, cast=False):
    n = len(ws)
    for j, (w, b) in enumerate(ws):
        if cast:
            v = v.astype(jnp.bfloat16)
        v = jnp.dot(v, w, preferred_element_type=jnp.float32) + b
        if j < n - 1:
            v = _leaky(v)
    return v


def _edge_body(ea_ref, *refs):
    out_ref = refs[-1]
    w = [r[...] for r in refs[:-1]]
    enc = [(w[0], w[1]), (w[2], w[3]), (w[4], w[5])]
    mcat = [(w[6], w[7]), (w[8], w[9]), (w[10], w[11])]
    z = _mlp(ea_ref[...], enc)
    mu = jnp.mean(z, axis=-1, keepdims=True)
    var = jnp.mean((z - mu) ** 2, axis=-1, keepdims=True)
    t = (z - mu) * lax.rsqrt(var + 1e-5)
    out_ref[...] = _mlp(t, mcat)


def _edge_kernel(edge_attr, weights, half, ne):
    full = [
        pl.BlockSpec(a.shape, lambda i, nd=a.ndim: (0,) * nd) for a in weights
    ]
    nsteps = ne // _TE
    return pl.pallas_call(
        _edge_body,
        grid=(nsteps,),
        in_specs=[
            pl.BlockSpec((_TE, 16), lambda i, h=half, n=nsteps: (i + h * n, 0))
        ]
        + full,
        out_specs=pl.BlockSpec((_TE, 128), lambda i: (i, 0)),
        out_shape=jax.ShapeDtypeStruct((ne, 128), jnp.float32),
        compiler_params=pltpu.CompilerParams(
            dimension_semantics=("arbitrary",)
        ),
    )(edge_attr, *weights)


def _make_scatter_body(ne, off):
    nch = ne // _CHUNK
    cpw = nch // _NW
    rem = nch % _NW
    nfull = cpw - (cpw % 2)

    def scatter_body(
        msg_hbm, dst_hbm, zeros_hbm, out_hbm, idx_v, rows_v, sem0, sem1, acc_sh
    ):
        c = lax.axis_index("c")
        s = lax.axis_index("s")
        wid = c * _NS + s
        r0 = s * _RPT
        sems = (sem0, sem1)

        def start(j, b):
            base = (wid + j * _NW) * _CHUNK
            pltpu.async_copy(
                dst_hbm.at[pl.ds(off + base, _CHUNK)], idx_v.at[b], sems[b]
            )
            pltpu.async_copy(
                msg_hbm.at[pl.ds(base, _CHUNK)], rows_v.at[b], sems[b]
            )

        def wait(b):
            pltpu.make_async_copy(
                dst_hbm.at[pl.ds(0, _CHUNK)], idx_v.at[b], sems[b]
            ).wait()
            pltpu.make_async_copy(
                msg_hbm.at[pl.ds(0, _CHUNK)], rows_v.at[b], sems[b]
            ).wait()

        def chunk_sync(j):
            base = (wid + j * _NW) * _CHUNK
            pltpu.sync_copy(dst_hbm.at[pl.ds(off + base, _CHUNK)], idx_v.at[0])
            pltpu.sync_copy(msg_hbm.at[pl.ds(base, _CHUNK)], rows_v.at[0])
            pltpu.sync_copy(rows_v.at[0], acc_sh.at[idx_v.at[0]], add=True)

        for b in range(2):
            start(b, b)

        pltpu.sync_copy(
            zeros_hbm.at[pl.ds(r0, _RPT)], acc_sh.at[pl.ds(r0, _RPT)]
        )
        plsc.subcore_barrier()

        def body(it, carry):
            j0 = it * 2
            for b in range(2):
                j = j0 + b
                wait(b)
                pltpu.sync_copy(rows_v.at[b], acc_sh.at[idx_v.at[b]], add=True)

                @pl.when(j + 2 < nfull)
                def _():
                    start(j + 2, b)

            return carry

        lax.fori_loop(0, nfull // 2, body, 0)

        for j in range(nfull, cpw):
            chunk_sync(j)

        @pl.when(wid < rem)
        def _():
            chunk_sync(cpw)

        plsc.subcore_barrier()
        pltpu.sync_copy(
            acc_sh.at[pl.ds(r0, _RPT)], out_hbm.at[c, pl.ds(r0, _RPT)]
        )

    return scatter_body


@functools.cache
def _build_scatter_kernel(ne, off):
    return functools.partial(
        pl.kernel,
        out_type=jax.ShapeDtypeStruct((_NC, _NPAD, 128), jnp.float32),
        mesh=plsc.VectorSubcoreMesh(
            core_axis_name="c", subcore_axis_name="s", num_cores=_NC
        ),
        scratch_types=[
            pltpu.VMEM((2, _CHUNK), jnp.int32),
            pltpu.VMEM((2, _CHUNK, 128), jnp.float32),
            pltpu.SemaphoreType.DMA,
            pltpu.SemaphoreType.DMA,
            pltpu.VMEM_SHARED((_NPAD, 128), jnp.float32),
        ],
    )(_make_scatter_body(ne, off))


def _scatter_kernel(msg, dst, zeros, off=0):
    return _build_scatter_kernel(msg.shape[0], off)(msg, dst, zeros)


def _node_body(nparts, x_ref, *refs):
    p_refs = refs[:nparts]
    out_ref = refs[-1]
    w = [r[...] for r in refs[nparts:-1]]
    enc = [(w[0], w[1]), (w[2], w[3]), (w[4], w[5])]
    eg, eb = w[6], w[7]
    dec = [(w[8], w[9]), (w[10], w[11]), (w[12], w[13])]
    y = _ln(_mlp(x_ref[...], enc), eg, eb)
    p = sum(pr[0] + pr[1] for pr in p_refs)
    aggrs = [
        lax.slice_in_dim(p, 0, 64, axis=1),
        lax.slice_in_dim(p, 64, 128, axis=1),
    ]
    k = 14
    for i in range(2):
        ng, nb = w[k], w[k + 1]
        upd = [(w[k + 2], w[k + 3]), (w[k + 4], w[k + 5]), (w[k + 6], w[k + 7])]
        og, ob = w[k + 8], w[k + 9]
        k += 10
        h = _ln(jnp.concatenate([y, aggrs[i]], axis=-1), ng, nb)
        y = y + _ln(_mlp(h, upd), og, ob)
    out_ref[...] = _mlp(y, dec)


def _node_kernel(x, partials_list, weights):
    full = [
        pl.BlockSpec(a.shape, lambda i, nd=a.ndim: (0,) * nd) for a in weights
    ]
    pspecs = [
        pl.BlockSpec((_NC, _TN, 128), lambda i: (0, i, 0))
        for _ in partials_list
    ]
    return pl.pallas_call(
        functools.partial(_node_body, len(partials_list)),
        grid=(_NN // _TN,),
        in_specs=[pl.BlockSpec((_TN, 128), lambda i: (i, 0))]
        + pspecs
        + full,
        out_specs=pl.BlockSpec((_TN, 128), lambda i: (i, 0)),
        out_shape=jax.ShapeDtypeStruct((_NN, 128), jnp.float32),
        compiler_params=pltpu.CompilerParams(
            dimension_semantics=("arbitrary",)
        ),
    )(x, *partials_list, *weights)


def _flat_mlp(ps, wdtype=jnp.float32):
    out = []
    for wmat, bvec in ps:
        out.append(jnp.transpose(wmat).astype(wdtype))
        out.append(jnp.reshape(bvec, (1, -1)))
    return out


def _flat_ln(p):
    g, b = p
    return [jnp.reshape(g, (1, -1)), jnp.reshape(b, (1, -1))]


def kernel(x, edge_attr, params, edge_index):
    dst = edge_index[1].astype(jnp.int32)

    m0, m1 = params['layers'][0]['msg'], params['layers'][1]['msg']
    g, bn = params['enc_edge']['ln']
    w1t = jnp.concatenate(
        [jnp.transpose(m0[0][0]), jnp.transpose(m1[0][0])], axis=1
    )
    b1c = jnp.concatenate([m0[0][1], m1[0][1]])
    w1f = g[:, None] * w1t
    b1f = jnp.reshape(bn @ w1t + b1c, (1, -1))

    def _blockdiag(a, b):
        za = jnp.zeros_like(a)
        return jnp.block([[a, za], [za, b]])

    mcat = [(w1f, b1f)]
    for j in (1, 2):
        wd = _blockdiag(jnp.transpose(m0[j][0]), jnp.transpose(m1[j][0]))
        bd = jnp.reshape(jnp.concatenate([m0[j][1], m1[j][1]]), (1, -1))
        mcat.append((wd, bd))

    edge_w = _flat_mlp(params['enc_edge']['mlp']) + [
        a for wb in mcat for a in wb
    ]
    node_w = (
        _flat_mlp(params['enc']['mlp'])
        + _flat_ln(params['enc']['ln'])
        + _flat_mlp(params['dec'])
    )
    for lp in params['layers']:
        node_w += (
            _flat_ln(lp['norm']) + _flat_mlp(lp['upd']) + _flat_ln(lp['outer_ln'])
        )

    zeros = jnp.zeros((_NPAD, 128), jnp.float32)
    nsplit = 4
    eh = _EN // nsplit
    parts = []
    for q in range(nsplit):
        msg_q = _edge_kernel(edge_attr, edge_w, q, eh)
        parts.append(_scatter_kernel(msg_q, dst, zeros, q * eh))
    return _node_kernel(x, parts, node_w)
